# baseline (speedup 1.0000x reference)
import functools

import jax
import jax.numpy as jnp
from jax import lax
from jax.experimental import pallas as pl
from jax.experimental.pallas import tpu as pltpu
from jax.experimental.pallas import tpu_sc as plsc

D = 64
SEQ = 200
BATCH = 1024
NC = 2
NS = 16
NW = NC * NS
BPW = BATCH // NW
SPC = 2
CHUNK = SPC * SEQ
NCHUNK = BPW // SPC
NGROUP = CHUNK // 16
PAD = D + 1
LN_EPS = 1e-5

IDX_SPLIT = ((0, 56), (56, 48), (104, 48), (152, 48))


def _emb_ln_kernel(ids_hbm, w_hbm, pos_hbm, gam_hbm, bet_hbm, out_hbm,
                   idx_v, pos_v, pbuf_a, pbuf_b, obuf, cbuf, gb_v,
                   gbx_v, gsem_a, gsem_b, osem):
    cid = lax.axis_index("c")
    sid = lax.axis_index("s")
    wid = sid * NC + cid
    wb = wid * BPW

    pltpu.sync_copy(ids_hbm.at[pl.ds(wb, BPW)], idx_v)
    pltpu.sync_copy(pos_hbm, pos_v)
    pltpu.sync_copy(gam_hbm, gb_v.at[0])
    pltpu.sync_copy(bet_hbm, gb_v.at[1])

    lanes = lax.broadcasted_iota(jnp.int32, (16,), 0)
    zero = jnp.zeros((16,), jnp.float32)
    zero_i = jnp.zeros((16,), jnp.int32)
    lanes_k = [lanes + (k * 16) for k in range(D // 16)]

    himask = jnp.full((16,), -65536, jnp.int32)

    def expand_gb(j, carry):
        cj = zero_i + j
        gv = plsc.load_gather(gb_v, [zero_i, cj])
        bv = plsc.load_gather(gb_v, [zero_i + 1, cj])
        gi = plsc.bitcast(gv, jnp.int32) & himask
        bi = lax.shift_right_logical(plsc.bitcast(bv, jnp.int32), 16)
        gbx_v[j] = gi | bi
        return carry

    lax.fori_loop(0, D, expand_gb, 0)

    def start_gather(c, buf, sem):
        for s in range(SPC):
            for k, (off, n) in enumerate(IDX_SPLIT):
                pltpu.async_copy(
                    w_hbm.at[idx_v.at[c * SPC + s, pl.ds(off, n)]],
                    buf.at[pl.ds(s * SEQ + off, n)],
                    sem.at[s * len(IDX_SPLIT) + k],
                )

    def drain(buf, sem):
        for s in range(SPC):
            for k, (off, n) in enumerate(IDX_SPLIT):
                pltpu.make_async_copy(
                    w_hbm.at[idx_v.at[s, pl.ds(off, n)]],
                    buf.at[pl.ds(s * SEQ + off, n)],
                    sem.at[s * len(IDX_SPLIT) + k],
                ).wait()

    def unpack_add(pbuf):
        @plsc.parallel_loop(0, CHUNK, step=1, unroll=8)
        def _(r):
            prow = lax.rem(r, SEQ)
            base = zero_i + r * PAD
            for k in range(D // 16):
                wv = pbuf[r, pl.ds(k * 16, 16)]
                pv = pos_v[prow, pl.ds(k * 16, 16)]
                plsc.store_scatter(cbuf, [base + lanes_k[k]], wv + pv)

    def compute():
        def group_body(g, gcarry):
            rowb = (g * 16 + lanes) * PAD

            @plsc.parallel_loop(0, D, step=1, unroll=16, carry=(zero, zero))
            def p1(j, acc):
                s_in, q_in = acc
                sv = plsc.load_gather(cbuf, [rowb + j])
                return (s_in + sv, q_in + sv * sv)

            s_acc, q_acc = p1
            mean = s_acc * (1.0 / 64.0)
            var = q_acc * (1.0 / 64.0) - mean * mean
            x = var + LN_EPS
            i = plsc.bitcast(x, jnp.int32)
            i = 0x5F3759DF - lax.shift_right_logical(i, 1)
            y = plsc.bitcast(i, jnp.float32)
            half = x * 0.5
            y = y * (1.5 - half * y * y)
            y = y * (1.5 - half * y * y)
            y = y * (1.5 - half * y * y)
            rstd = y

            @plsc.parallel_loop(0, D, step=1, unroll=8)
            def p2(j):
                ii = rowb + j
                sv = plsc.load_gather(cbuf, [ii])
                gv = gbx_v[j]
                gf = plsc.bitcast(gv & himask, jnp.float32)
                bf = plsc.bitcast(lax.shift_left(gv, 16), jnp.float32)
                a = rstd * gf
                b = bf - mean * a
                plsc.store_scatter(cbuf, [ii], sv * a + b)

            return gcarry

        lax.fori_loop(0, NGROUP, group_body, 0)

    def pack():
        @plsc.parallel_loop(0, CHUNK, step=1, unroll=8)
        def _(r):
            base = zero_i + r * PAD
            for k in range(D // 16):
                ov = plsc.load_gather(cbuf, [base + lanes_k[k]])
                obuf[r, pl.ds(k * 16, 16)] = ov

    def start_out(c):
        for s in range(SPC):
            pltpu.async_copy(
                obuf.at[pl.ds(s * SEQ, SEQ)],
                out_hbm.at[wb + c * SPC + s],
                osem,
            )

    def drain_out():
        for s in range(SPC):
            pltpu.make_async_copy(
                obuf.at[pl.ds(s * SEQ, SEQ)],
                out_hbm.at[wb + s],
                osem,
            ).wait()

    def process(c, pbuf, gsem, first):
        drain(pbuf, gsem)
        unpack_add(pbuf)

        @pl.when(c + 2 < NCHUNK)
        def _():
            start_gather(c + 2, pbuf, gsem)

        compute()

        @pl.when(jnp.logical_not(first))
        def _():
            drain_out()

        pack()
        start_out(c)

    start_gather(0, pbuf_a, gsem_a)
    start_gather(1, pbuf_b, gsem_b)

    def pair_body(i, carry):
        ca = i * 2
        process(ca, pbuf_a, gsem_a, i == 0)
        process(ca + 1, pbuf_b, gsem_b, False)
        return carry

    lax.fori_loop(0, NCHUNK // 2, pair_body, 0)
    drain_out()


@functools.partial(
    pl.kernel,
    out_type=jax.ShapeDtypeStruct((BATCH, SEQ, D), jnp.float32),
    mesh=plsc.VectorSubcoreMesh(core_axis_name="c", subcore_axis_name="s"),
    scratch_types=[
        pltpu.VMEM((BPW, SEQ), jnp.int32),
        pltpu.VMEM((SEQ, D), jnp.float32),
        pltpu.VMEM((CHUNK, D), jnp.float32),
        pltpu.VMEM((CHUNK, D), jnp.float32),
        pltpu.VMEM((CHUNK, D), jnp.float32),
        pltpu.VMEM((CHUNK * PAD,), jnp.float32),
        pltpu.VMEM((2, D), jnp.float32),
        pltpu.VMEM((D, 16), jnp.int32),
        pltpu.SemaphoreType.DMA((8,)),
        pltpu.SemaphoreType.DMA((8,)),
        pltpu.SemaphoreType.DMA,
    ],
    compiler_params=pltpu.CompilerParams(
        needs_layout_passes=False, use_tc_tiling_on_sc=False,
        disable_bounds_checks=True),
)
def _emb_ln(ids, w, pos, gam, bet, out, idx_v, pos_v, pbuf_a, pbuf_b, obuf,
            cbuf, gb_v, gbx_v, gsem_a, gsem_b, osem):
    _emb_ln_kernel(ids, w, pos, gam, bet, out, idx_v, pos_v, pbuf_a, pbuf_b,
                   obuf, cbuf, gb_v, gbx_v, gsem_a, gsem_b, osem)


def kernel(input_ids, W_word, pos_table, ln_gamma, ln_beta):
    ids = input_ids.astype(jnp.int32)
    pos = pos_table[:SEQ]
    return _emb_ln(ids, W_word, pos, ln_gamma, ln_beta)
